# baseline (speedup 1.0000x reference)
import functools

import jax
import jax.numpy as jnp
from jax import lax
from jax.experimental import pallas as pl
from jax.experimental.pallas import tpu as pltpu
from jax.experimental.pallas import tpu_sc as plsc

K_NB = 16
ALPHA = 0.5
PRE = 128
RANK_CHUNK = 256
RCH = 8


def _matmul_body(feat_ref, ker_ref, attn_ref, sup_ref, imp_ref):
    f = feat_ref[0]
    sup_ref[0] = jnp.dot(f, ker_ref[...], preferred_element_type=jnp.float32)
    imp_ref[0] = jnp.dot(f, attn_ref[...], preferred_element_type=jnp.float32)


def _rank_body(imp_ref, rank_ref):
    jb = pl.program_id(1)
    n = imp_ref.shape[2]
    base = jb * RANK_CHUNK
    row = imp_ref[0, 0, :]
    vj = imp_ref[0, 0, pl.ds(base, RANK_CHUNK)]
    vj2 = vj[:, None]
    jidx = base + lax.broadcasted_iota(jnp.int32, (RANK_CHUNK, 1), 0)
    jp = lax.broadcasted_iota(jnp.int32, (RANK_CHUNK, n), 1)
    rowb = row[None, :]
    before = (rowb > vj2) | ((rowb == vj2) & (jp < jidx))
    rank_ref[0, 0, :] = jnp.sum(before.astype(jnp.int32), axis=1)


def _pack_body(adj_ref, pmat_ref, out_ref):
    h = jnp.dot(adj_ref[...].astype(jnp.bfloat16), pmat_ref[...],
                preferred_element_type=jnp.float32)
    nw = out_ref.shape[1]
    lo = h[:, :nw].astype(jnp.int32)
    hi = h[:, nw:].astype(jnp.int32)
    out_ref[...] = lo | (hi << 16)


def _sc_body(nc, ns, n, b_total, d,
             padj_hbm, feat_hbm, sup_hbm, imp_hbm, rank_hbm, bias_hbm,
             out_hbm,
             perm_v, simp_v, rank_v, impb_v, bias_v, jsel_v, wsel_v,
             slots_v, pch0_v, pch1_v,
             jr0_v, jr1_v, nb0_v, nb1_v,
             fr0_v, fr1_v, or0_v, or1_v,
             sem_p0, sem_p1, sem_s0, sem_s1, sem_f0, sem_f1,
             sem_o0, sem_o1):
    nw = n // 32
    rows_per = n // (nc * ns)
    cid = lax.axis_index("c")
    sid = lax.axis_index("s")
    wid = sid * nc + cid
    lanes = lax.iota(jnp.int32, 16)
    pch = (pch0_v, pch1_v)
    sem_p = (sem_p0, sem_p1)
    jrb = (jr0_v, jr1_v)
    nbb = (nb0_v, nb1_v)
    frb = (fr0_v, fr1_v)
    orb = (or0_v, or1_v)
    sem_s = (sem_s0, sem_s1)
    sem_f = (sem_f0, sem_f1)
    sem_o = (sem_o0, sem_o1)

    pltpu.sync_copy(bias_hbm, bias_v)

    def batch_fn(b, carry0):
        row0 = wid * rows_per
        pltpu.sync_copy(imp_hbm.at[b], impb_v)
        pltpu.sync_copy(rank_hbm.at[b], rank_v)

        def build(i, carry):
            sl = pl.ds(i * 16, 16)
            r = rank_v[sl]
            jv = i * 16 + lanes
            plsc.store_scatter(perm_v, [r], jv)
            plsc.store_scatter(simp_v, [r], impb_v[sl])
            return carry
        lax.fori_loop(0, n // 16, build, jnp.int32(0))

        row0 = wid * rows_per

        def scan128(prow_ref, c_base, found):
            for v in range(PRE // 16):
                pv = perm_v[pl.ds(c_base + v * 16, 16)]
                word = plsc.load_gather(prow_ref, [pv >> 5])
                bit = (word >> (pv & 31)) & 1
                m = bit != 0
                mi = bit
                cpos = plsc.cumsum(mi) + found
                sel = jnp.logical_and(m, cpos <= K_NB)
                slot = jnp.clip(cpos - 1, 0, 15)
                cvec = c_base + v * 16 + lanes
                plsc.store_scatter(slots_v, [slot], cvec, mask=sel)
                found = found + jnp.sum(mi)
            return found

        def select_row(prow_ref, ri):
            slots_v[...] = jnp.zeros((16,), jnp.int32)

            def cond(st):
                c_base, fnd = st
                return jnp.logical_and(fnd < K_NB, c_base < n)

            def more(st):
                c_base, fnd = st
                fnd = scan128(prow_ref, c_base, fnd)
                return (c_base + PRE, fnd)

            _, found = lax.while_loop(cond, more,
                                      (jnp.int32(0), jnp.int32(0)))

            c16 = slots_v[...]
            jv = plsc.load_gather(perm_v, [c16])
            vals = plsc.load_gather(simp_v, [c16])
            t0 = found == 0
            valid = jnp.logical_or(lanes < found, t0)
            jv = jnp.where(t0, lanes, jv)
            vals = jnp.where(valid, jnp.where(t0, 0.0, vals), -3.0e38)
            mx = jnp.max(vals)
            e = jnp.where(valid, jnp.exp(vals - mx), 0.0)
            s = jnp.sum(e)
            jsel_v[pl.ds(ri * 16, 16)] = jv + b * n
            wsel_v[pl.ds(ri * 16, 16)] = e / s

        nch = rows_per // RCH
        row0 = wid * rows_per

        def sel_chunk(g, carry):
            for sub in range(2):
                buf = pch[sub]

                @pl.when(2 * g + sub + 1 < nch)
                def _():
                    nxt = row0 + (2 * g + sub + 1) * RCH
                    pltpu.async_copy(padj_hbm.at[pl.ds(nxt, RCH)],
                                     pch[1 - sub], sem_p[1 - sub])
                pltpu.make_async_copy(padj_hbm.at[pl.ds(0, RCH)],
                                      buf, sem_p[sub]).wait()
                for r in range(RCH):
                    select_row(buf.at[r], (2 * g + sub) * RCH + r)
            return carry

        pltpu.async_copy(padj_hbm.at[pl.ds(row0, RCH)], pch0_v, sem_p0)
        lax.fori_loop(0, nch // 2, sel_chunk, jnp.int32(0))

        def issue(ri, buf):
            jrb[buf][...] = jsel_v[pl.ds(ri * 16, 16)]
            pltpu.async_copy(sup_hbm.at[jrb[buf]], nbb[buf], sem_s[buf])
            pltpu.async_copy(
                feat_hbm.at[pl.ds((b * n + row0 + ri) * d, d)],
                frb[buf], sem_f[buf])

        def agg_row(ri, buf):
            pltpu.make_async_copy(sup_hbm.at[jrb[buf]], nbb[buf],
                                  sem_s[buf]).wait()
            pltpu.make_async_copy(feat_hbm.at[pl.ds(0, d)], frb[buf],
                                  sem_f[buf]).wait()
            w16 = wsel_v[pl.ds(ri * 16, 16)]
            wk = [w16[jnp.full((16,), k, jnp.int32)] for k in range(K_NB)]
            nb_v = nbb[buf]
            frow_v = frb[buf]
            orow_v = orb[buf]

            @pl.when(ri >= 2)
            def _():
                pltpu.make_async_copy(orow_v, out_hbm.at[pl.ds(0, d)],
                                      sem_o[buf]).wait()
            for dv in range(d // 16):
                dsl = pl.ds(dv * 16, 16)
                accs = [wk[c] * nb_v[c, dsl] for c in range(4)]
                for k in range(4, K_NB):
                    c = k % 4
                    accs[c] = accs[c] + wk[k] * nb_v[k, dsl]
                acc = (accs[0] + accs[1]) + (accs[2] + accs[3])
                o = ALPHA * frow_v[dsl] + (1.0 - ALPHA) * acc + bias_v[dsl]
                orow_v[dsl] = jnp.maximum(o, 0.0)

            @pl.when(ri + 2 < rows_per)
            def _():
                issue(ri + 2, buf)
            pltpu.async_copy(
                orow_v,
                out_hbm.at[pl.ds((b * n + row0 + ri) * d, d)],
                sem_o[buf])

        issue(jnp.int32(0), 0)
        issue(jnp.int32(1), 1)

        def agg_pair(g, carry):
            agg_row(2 * g, 0)
            agg_row(2 * g + 1, 1)
            return carry
        lax.fori_loop(0, rows_per // 2, agg_pair, jnp.int32(0))

        pltpu.make_async_copy(or0_v, out_hbm.at[pl.ds(0, d)], sem_o0).wait()
        pltpu.make_async_copy(or1_v, out_hbm.at[pl.ds(0, d)], sem_o1).wait()
        return carry0

    lax.fori_loop(0, b_total, batch_fn, jnp.int32(0))


def _tc_stage(features, kernel, attn_kernel):
    b_total, n, f = features.shape
    d = kernel.shape[1]
    bn = 512

    sup, imp = pl.pallas_call(
        _matmul_body,
        grid=(b_total, n // bn),
        in_specs=[
            pl.BlockSpec((1, bn, f), lambda b, nb: (b, nb, 0)),
            pl.BlockSpec((f, d), lambda b, nb: (0, 0)),
            pl.BlockSpec((f, 1), lambda b, nb: (0, 0)),
        ],
        out_specs=[
            pl.BlockSpec((1, bn, d), lambda b, nb: (b, nb, 0)),
            pl.BlockSpec((1, bn, 1), lambda b, nb: (b, nb, 0)),
        ],
        out_shape=[
            jax.ShapeDtypeStruct((b_total, n, d), jnp.float32),
            jax.ShapeDtypeStruct((b_total, n, 1), jnp.float32),
        ],
    )(features, kernel, attn_kernel)

    nchunks = n // RANK_CHUNK
    imp3 = imp.reshape(b_total, 1, n)
    rank = pl.pallas_call(
        _rank_body,
        grid=(b_total, nchunks),
        in_specs=[pl.BlockSpec((1, 1, n), lambda b, jb: (b, 0, 0))],
        out_specs=pl.BlockSpec((1, 1, RANK_CHUNK),
                               lambda b, jb: (b * nchunks + jb, 0, 0)),
        out_shape=jax.ShapeDtypeStruct((b_total * nchunks, 1, RANK_CHUNK),
                                       jnp.int32),
    )(imp3).reshape(b_total, n)
    return sup, imp, rank


def _pack_adj(adj):
    n = adj.shape[0]
    nw = n // 32
    bn = 512
    col = jnp.arange(n)
    word = col // 32
    bit = col % 32
    tgt = jnp.where(bit < 16, word, nw + word)
    pmat = (jnp.zeros((n, 2 * nw), jnp.float32)
            .at[col, tgt].set(jnp.exp2((bit % 16).astype(jnp.float32))))
    pmat = pmat.astype(jnp.bfloat16)
    return pl.pallas_call(
        _pack_body,
        grid=(n // bn,),
        in_specs=[
            pl.BlockSpec((bn, n), lambda i: (i, 0)),
            pl.BlockSpec((n, 2 * nw), lambda i: (0, 0)),
        ],
        out_specs=pl.BlockSpec((bn, nw), lambda i: (i, 0)),
        out_shape=jax.ShapeDtypeStruct((n, nw), jnp.int32),
    )(adj, pmat)


def _sc_stage(adj, features, sup, imp, rank, bias):
    b_total, n, f = features.shape
    d = sup.shape[2]
    padj = _pack_adj(adj)
    try:
        info = plsc.get_sparse_core_info()
        nc, ns = info.num_cores, info.num_subcores
    except Exception:
        nc, ns = 2, 16

    nwords = n // 32
    body = functools.partial(_sc_body, nc, ns, n, b_total, d)
    sc = pl.kernel(
        body,
        out_type=jax.ShapeDtypeStruct((b_total * n * d,), jnp.float32),
        mesh=plsc.VectorSubcoreMesh(core_axis_name="c", subcore_axis_name="s"),
        compiler_params=pltpu.CompilerParams(needs_layout_passes=False),
        scratch_types=[
            pltpu.VMEM((n,), jnp.int32),
            pltpu.VMEM((n,), jnp.float32),
            pltpu.VMEM((n,), jnp.int32),
            pltpu.VMEM((n,), jnp.float32),
            pltpu.VMEM((d,), jnp.float32),
            pltpu.VMEM((n // (nc * ns) * 16,), jnp.int32),
            pltpu.VMEM((n // (nc * ns) * 16,), jnp.float32),
            pltpu.VMEM((16,), jnp.int32),
            pltpu.VMEM((RCH, nwords), jnp.int32),
            pltpu.VMEM((RCH, nwords), jnp.int32),
            pltpu.VMEM((16,), jnp.int32),
            pltpu.VMEM((16,), jnp.int32),
            pltpu.VMEM((16, d), jnp.float32),
            pltpu.VMEM((16, d), jnp.float32),
            pltpu.VMEM((d,), jnp.float32),
            pltpu.VMEM((d,), jnp.float32),
            pltpu.VMEM((d,), jnp.float32),
            pltpu.VMEM((d,), jnp.float32),
            pltpu.SemaphoreType.DMA, pltpu.SemaphoreType.DMA,
            pltpu.SemaphoreType.DMA, pltpu.SemaphoreType.DMA,
            pltpu.SemaphoreType.DMA, pltpu.SemaphoreType.DMA,
            pltpu.SemaphoreType.DMA, pltpu.SemaphoreType.DMA,
        ],
    )
    out_flat = sc(padj,
                  features.reshape(b_total * n * f),
                  sup.reshape(b_total * n, d),
                  imp.reshape(b_total, n),
                  rank,
                  bias)
    return out_flat.reshape(b_total, n, d)


def kernel(adj, features, attn_kernel, kernel, bias):
    sup, imp, rank = _tc_stage(features, kernel, attn_kernel)
    return _sc_stage(adj, features, sup, imp, rank, bias)


# speedup vs baseline: 1.1025x; 1.1025x over previous
import functools

import jax
import jax.numpy as jnp
from jax import lax
from jax.experimental import pallas as pl
from jax.experimental.pallas import tpu as pltpu
from jax.experimental.pallas import tpu_sc as plsc

K_NB = 16
ALPHA = 0.5
PRE = 128
RANK_CHUNK = 256
RCH = 8


def _matmul_body(feat_ref, ker_ref, attn_ref, sup_ref, imp_ref):
    f = feat_ref[0]
    sup_ref[0] = jnp.dot(f, ker_ref[...], preferred_element_type=jnp.float32)
    imp_ref[0] = jnp.dot(f, attn_ref[...], preferred_element_type=jnp.float32)


def _rank_body(imp_ref, rank_ref):
    jb = pl.program_id(1)
    n = imp_ref.shape[2]
    base = jb * RANK_CHUNK
    row = imp_ref[0, 0, :]
    vj = imp_ref[0, 0, pl.ds(base, RANK_CHUNK)]
    vj2 = vj[:, None]
    jidx = base + lax.broadcasted_iota(jnp.int32, (RANK_CHUNK, 1), 0)
    jp = lax.broadcasted_iota(jnp.int32, (RANK_CHUNK, n), 1)
    rowb = row[None, :]
    before = (rowb > vj2) | ((rowb == vj2) & (jp < jidx))
    rank_ref[0, 0, :] = jnp.sum(before.astype(jnp.int32), axis=1)


def _pack_body(adj_ref, pmat_ref, out_ref):
    h = jnp.dot(adj_ref[...].astype(jnp.bfloat16), pmat_ref[...],
                preferred_element_type=jnp.float32)
    nw = out_ref.shape[1]
    lo = h[:, :nw].astype(jnp.int32)
    hi = h[:, nw:].astype(jnp.int32)
    out_ref[...] = lo | (hi << 16)


def _sc_body(nc, ns, n, b_total, d,
             padj_hbm, feat_hbm, sup_hbm, imp_hbm, rank_hbm, bias_hbm,
             out_hbm,
             perm_v, simp_v, rank_v, impb_v, bias_v, jsel_v, wsel_v,
             slots_v, pch0_v, pch1_v,
             jr0_v, jr1_v, nb0_v, nb1_v,
             fr0_v, fr1_v, or0_v, or1_v,
             sem_p0, sem_p1, sem_s0, sem_s1, sem_f0, sem_f1,
             sem_o0, sem_o1):
    nw = n // 32
    rows_per = n // (nc * ns)
    cid = lax.axis_index("c")
    sid = lax.axis_index("s")
    wid = sid * nc + cid
    lanes = lax.iota(jnp.int32, 16)
    pch = (pch0_v, pch1_v)
    sem_p = (sem_p0, sem_p1)
    jrb = (jr0_v, jr1_v)
    nbb = (nb0_v, nb1_v)
    frb = (fr0_v, fr1_v)
    orb = (or0_v, or1_v)
    sem_s = (sem_s0, sem_s1)
    sem_f = (sem_f0, sem_f1)
    sem_o = (sem_o0, sem_o1)

    pltpu.sync_copy(bias_hbm, bias_v)

    def batch_fn(b, carry0):
        row0 = wid * rows_per
        pltpu.sync_copy(imp_hbm.at[b], impb_v)
        pltpu.sync_copy(rank_hbm.at[b], rank_v)

        def build(i, carry):
            sl = pl.ds(i * 16, 16)
            r = rank_v[sl]
            jv = i * 16 + lanes
            plsc.store_scatter(perm_v, [r], jv)
            plsc.store_scatter(simp_v, [r], impb_v[sl])
            return carry
        lax.fori_loop(0, n // 16, build, jnp.int32(0))

        row0 = wid * rows_per

        def scan128(prow_ref, c_base, found):
            for v in range(PRE // 16):
                pv = perm_v[pl.ds(c_base + v * 16, 16)]
                word = plsc.load_gather(prow_ref, [pv >> 5])
                bit = (word >> (pv & 31)) & 1
                m = bit != 0
                mi = bit
                cpos = plsc.cumsum(mi) + found
                sel = jnp.logical_and(m, cpos <= K_NB)
                slot = jnp.clip(cpos - 1, 0, 15)
                cvec = c_base + v * 16 + lanes
                plsc.store_scatter(slots_v, [slot], cvec, mask=sel)
                found = found + jnp.sum(mi)
            return found

        def select_row(prow_ref, ri):
            slots_v[...] = jnp.zeros((16,), jnp.int32)

            def cond(st):
                c_base, fnd = st
                return jnp.logical_and(fnd < K_NB, c_base < n)

            def more(st):
                c_base, fnd = st
                fnd = scan128(prow_ref, c_base, fnd)
                return (c_base + PRE, fnd)

            _, found = lax.while_loop(cond, more,
                                      (jnp.int32(0), jnp.int32(0)))

            c16 = slots_v[...]
            jv = plsc.load_gather(perm_v, [c16])
            vals = plsc.load_gather(simp_v, [c16])
            t0 = found == 0
            valid = jnp.logical_or(lanes < found, t0)
            jv = jnp.where(t0, lanes, jv)
            vals = jnp.where(valid, jnp.where(t0, 0.0, vals), -3.0e38)
            mx = jnp.max(vals)
            e = jnp.where(valid, jnp.exp(vals - mx), 0.0)
            s = jnp.sum(e)
            jsel_v[pl.ds(ri * 16, 16)] = jv + b * n
            wsel_v[pl.ds(ri * 16, 16)] = e / s

        nch = rows_per // RCH
        row0 = wid * rows_per

        def sel_chunk(g, carry):
            for sub in range(2):
                buf = pch[sub]

                @pl.when(2 * g + sub + 1 < nch)
                def _():
                    nxt = row0 + (2 * g + sub + 1) * RCH
                    pltpu.async_copy(padj_hbm.at[pl.ds(nxt, RCH)],
                                     pch[1 - sub], sem_p[1 - sub])
                pltpu.make_async_copy(padj_hbm.at[pl.ds(0, RCH)],
                                      buf, sem_p[sub]).wait()
                for r in range(RCH):
                    select_row(buf.at[r], (2 * g + sub) * RCH + r)
            return carry

        pltpu.async_copy(padj_hbm.at[pl.ds(row0, RCH)], pch0_v, sem_p0)
        lax.fori_loop(0, nch // 2, sel_chunk, jnp.int32(0))

        def issue(ri, buf):
            jrb[buf][...] = jsel_v[pl.ds(ri * 16, 16)]
            pltpu.async_copy(sup_hbm.at[jrb[buf]], nbb[buf], sem_s[buf])
            pltpu.async_copy(feat_hbm.at[b * n + row0 + ri],
                             frb[buf], sem_f[buf])

        def agg_row(ri, buf):
            pltpu.make_async_copy(sup_hbm.at[jrb[buf]], nbb[buf],
                                  sem_s[buf]).wait()
            pltpu.make_async_copy(feat_hbm.at[0], frb[buf],
                                  sem_f[buf]).wait()
            w16 = wsel_v[pl.ds(ri * 16, 16)]
            wk = [w16[jnp.full((16,), k, jnp.int32)] for k in range(K_NB)]
            nb_v = nbb[buf]
            frow_v = frb[buf]
            orow_v = orb[buf]

            @pl.when(ri >= 2)
            def _():
                pltpu.make_async_copy(orow_v, out_hbm.at[0],
                                      sem_o[buf]).wait()
            for dv in range(d // 16):
                dsl = pl.ds(dv * 16, 16)
                accs = [wk[c] * nb_v[c, dsl] for c in range(4)]
                for k in range(4, K_NB):
                    c = k % 4
                    accs[c] = accs[c] + wk[k] * nb_v[k, dsl]
                acc = (accs[0] + accs[1]) + (accs[2] + accs[3])
                o = ALPHA * frow_v[dsl] + (1.0 - ALPHA) * acc + bias_v[dsl]
                orow_v[dsl] = jnp.maximum(o, 0.0)

            @pl.when(ri + 2 < rows_per)
            def _():
                issue(ri + 2, buf)
            pltpu.async_copy(orow_v, out_hbm.at[b * n + row0 + ri],
                             sem_o[buf])

        issue(jnp.int32(0), 0)
        issue(jnp.int32(1), 1)

        def agg_pair(g, carry):
            agg_row(2 * g, 0)
            agg_row(2 * g + 1, 1)
            return carry
        lax.fori_loop(0, rows_per // 2, agg_pair, jnp.int32(0))

        pltpu.make_async_copy(or0_v, out_hbm.at[0], sem_o0).wait()
        pltpu.make_async_copy(or1_v, out_hbm.at[0], sem_o1).wait()
        return carry0

    lax.fori_loop(0, b_total, batch_fn, jnp.int32(0))


def _tc_stage(features, kernel, attn_kernel):
    b_total, n, f = features.shape
    d = kernel.shape[1]
    bn = 512

    sup, imp = pl.pallas_call(
        _matmul_body,
        grid=(b_total, n // bn),
        in_specs=[
            pl.BlockSpec((1, bn, f), lambda b, nb: (b, nb, 0)),
            pl.BlockSpec((f, d), lambda b, nb: (0, 0)),
            pl.BlockSpec((f, 1), lambda b, nb: (0, 0)),
        ],
        out_specs=[
            pl.BlockSpec((1, bn, d), lambda b, nb: (b, nb, 0)),
            pl.BlockSpec((1, bn, 1), lambda b, nb: (b, nb, 0)),
        ],
        out_shape=[
            jax.ShapeDtypeStruct((b_total, n, d), jnp.float32),
            jax.ShapeDtypeStruct((b_total, n, 1), jnp.float32),
        ],
    )(features, kernel, attn_kernel)

    nchunks = n // RANK_CHUNK
    imp3 = imp.reshape(b_total, 1, n)
    rank = pl.pallas_call(
        _rank_body,
        grid=(b_total, nchunks),
        in_specs=[pl.BlockSpec((1, 1, n), lambda b, jb: (b, 0, 0))],
        out_specs=pl.BlockSpec((1, 1, RANK_CHUNK),
                               lambda b, jb: (b * nchunks + jb, 0, 0)),
        out_shape=jax.ShapeDtypeStruct((b_total * nchunks, 1, RANK_CHUNK),
                                       jnp.int32),
    )(imp3).reshape(b_total, n)
    return sup, imp, rank


def _pack_adj(adj):
    n = adj.shape[0]
    nw = n // 32
    bn = 512
    col = jnp.arange(n)
    word = col // 32
    bit = col % 32
    tgt = jnp.where(bit < 16, word, nw + word)
    pmat = (jnp.zeros((n, 2 * nw), jnp.float32)
            .at[col, tgt].set(jnp.exp2((bit % 16).astype(jnp.float32))))
    pmat = pmat.astype(jnp.bfloat16)
    return pl.pallas_call(
        _pack_body,
        grid=(n // bn,),
        in_specs=[
            pl.BlockSpec((bn, n), lambda i: (i, 0)),
            pl.BlockSpec((n, 2 * nw), lambda i: (0, 0)),
        ],
        out_specs=pl.BlockSpec((bn, nw), lambda i: (i, 0)),
        out_shape=jax.ShapeDtypeStruct((n, nw), jnp.int32),
    )(adj, pmat)


def _sc_stage(adj, features, sup, imp, rank, bias):
    b_total, n, f = features.shape
    d = sup.shape[2]
    padj = _pack_adj(adj)
    try:
        info = plsc.get_sparse_core_info()
        nc, ns = info.num_cores, info.num_subcores
    except Exception:
        nc, ns = 2, 16

    nwords = n // 32
    body = functools.partial(_sc_body, nc, ns, n, b_total, d)
    sc = pl.kernel(
        body,
        out_type=jax.ShapeDtypeStruct((b_total * n, d), jnp.float32),
        mesh=plsc.VectorSubcoreMesh(core_axis_name="c", subcore_axis_name="s"),
        compiler_params=pltpu.CompilerParams(needs_layout_passes=False),
        scratch_types=[
            pltpu.VMEM((n,), jnp.int32),
            pltpu.VMEM((n,), jnp.float32),
            pltpu.VMEM((n,), jnp.int32),
            pltpu.VMEM((n,), jnp.float32),
            pltpu.VMEM((d,), jnp.float32),
            pltpu.VMEM((n // (nc * ns) * 16,), jnp.int32),
            pltpu.VMEM((n // (nc * ns) * 16,), jnp.float32),
            pltpu.VMEM((16,), jnp.int32),
            pltpu.VMEM((RCH, nwords), jnp.int32),
            pltpu.VMEM((RCH, nwords), jnp.int32),
            pltpu.VMEM((16,), jnp.int32),
            pltpu.VMEM((16,), jnp.int32),
            pltpu.VMEM((16, d), jnp.float32),
            pltpu.VMEM((16, d), jnp.float32),
            pltpu.VMEM((d,), jnp.float32),
            pltpu.VMEM((d,), jnp.float32),
            pltpu.VMEM((d,), jnp.float32),
            pltpu.VMEM((d,), jnp.float32),
            pltpu.SemaphoreType.DMA, pltpu.SemaphoreType.DMA,
            pltpu.SemaphoreType.DMA, pltpu.SemaphoreType.DMA,
            pltpu.SemaphoreType.DMA, pltpu.SemaphoreType.DMA,
            pltpu.SemaphoreType.DMA, pltpu.SemaphoreType.DMA,
        ],
    )
    out_flat = sc(padj,
                  features.reshape(b_total * n, f),
                  sup.reshape(b_total * n, d),
                  imp.reshape(b_total, n),
                  rank,
                  bias)
    return out_flat.reshape(b_total, n, d)


def kernel(adj, features, attn_kernel, kernel, bias):
    sup, imp, rank = _tc_stage(features, kernel, attn_kernel)
    return _sc_stage(adj, features, sup, imp, rank, bias)


# speedup vs baseline: 1.1691x; 1.0604x over previous
import functools

import jax
import jax.numpy as jnp
from jax import lax
from jax.experimental import pallas as pl
from jax.experimental.pallas import tpu as pltpu
from jax.experimental.pallas import tpu_sc as plsc

K_NB = 16
ALPHA = 0.5
PRE = 64
RANK_CHUNK = 256
RCH = 8


def _matmul_body(feat_ref, ker_ref, attn_ref, sup_ref, imp_ref):
    f = feat_ref[0]
    sup_ref[0] = jnp.dot(f, ker_ref[...], preferred_element_type=jnp.float32)
    imp_ref[0] = jnp.dot(f, attn_ref[...], preferred_element_type=jnp.float32)


def _rank_body(imp_ref, rank_ref):
    jb = pl.program_id(1)
    n = imp_ref.shape[2]
    base = jb * RANK_CHUNK
    row = imp_ref[0, 0, :]
    vj = imp_ref[0, 0, pl.ds(base, RANK_CHUNK)]
    vj2 = vj[:, None]
    jidx = base + lax.broadcasted_iota(jnp.int32, (RANK_CHUNK, 1), 0)
    jp = lax.broadcasted_iota(jnp.int32, (RANK_CHUNK, n), 1)
    rowb = row[None, :]
    before = (rowb > vj2) | ((rowb == vj2) & (jp < jidx))
    rank_ref[0, 0, :] = jnp.sum(before.astype(jnp.int32), axis=1)


def _pack_body(adj_ref, pmat_ref, out_ref):
    h = jnp.dot(adj_ref[...].astype(jnp.bfloat16), pmat_ref[...],
                preferred_element_type=jnp.float32)
    nw = out_ref.shape[1]
    lo = h[:, :nw].astype(jnp.int32)
    hi = h[:, nw:].astype(jnp.int32)
    out_ref[...] = lo | (hi << 16)


def _sc_body(nc, ns, n, b_total, d,
             padj_hbm, feat_hbm, sup_hbm, imp_hbm, rank_hbm, bias_hbm,
             out_hbm,
             perm_v, simp_v, rank_v, impb_v, bias_v, jsel_v, wsel_v,
             slots_v, pch0_v, pch1_v,
             jr0_v, jr1_v, nb0_v, nb1_v,
             fr0_v, fr1_v, or0_v, or1_v,
             sem_p0, sem_p1, sem_s0, sem_s1, sem_f0, sem_f1,
             sem_o0, sem_o1):
    nw = n // 32
    rows_per = n // (nc * ns)
    cid = lax.axis_index("c")
    sid = lax.axis_index("s")
    wid = sid * nc + cid
    lanes = lax.iota(jnp.int32, 16)
    pch = (pch0_v, pch1_v)
    sem_p = (sem_p0, sem_p1)
    jrb = (jr0_v, jr1_v)
    nbb = (nb0_v, nb1_v)
    frb = (fr0_v, fr1_v)
    orb = (or0_v, or1_v)
    sem_s = (sem_s0, sem_s1)
    sem_f = (sem_f0, sem_f1)
    sem_o = (sem_o0, sem_o1)

    pltpu.sync_copy(bias_hbm, bias_v)

    def batch_fn(b, carry0):
        row0 = wid * rows_per
        pltpu.sync_copy(imp_hbm.at[b], impb_v)
        pltpu.sync_copy(rank_hbm.at[b], rank_v)

        def build(i, carry):
            sl = pl.ds(i * 16, 16)
            r = rank_v[sl]
            jv = i * 16 + lanes
            plsc.store_scatter(perm_v, [r], jv)
            plsc.store_scatter(simp_v, [r], impb_v[sl])
            return carry
        lax.fori_loop(0, n // 16, build, jnp.int32(0))

        row0 = wid * rows_per

        def scan128(prow_ref, c_base, found):
            for v in range(PRE // 16):
                pv = perm_v[pl.ds(c_base + v * 16, 16)]
                word = plsc.load_gather(prow_ref, [pv >> 5])
                bit = (word >> (pv & 31)) & 1
                m = bit != 0
                mi = bit
                cpos = plsc.cumsum(mi) + found
                sel = jnp.logical_and(m, cpos <= K_NB)
                slot = jnp.clip(cpos - 1, 0, 15)
                cvec = c_base + v * 16 + lanes
                plsc.store_scatter(slots_v, [slot], cvec, mask=sel)
                found = cpos[15]
            return found

        def select_row(prow_ref, ri):
            slots_v[...] = jnp.zeros((16,), jnp.int32)

            def cond(st):
                c_base, fnd = st
                return jnp.logical_and(fnd < K_NB, c_base < n)

            def more(st):
                c_base, fnd = st
                fnd = scan128(prow_ref, c_base, fnd)
                return (c_base + PRE, fnd)

            _, found = lax.while_loop(cond, more,
                                      (jnp.int32(0), jnp.int32(0)))

            c16 = slots_v[...]
            jv = plsc.load_gather(perm_v, [c16])
            vals = plsc.load_gather(simp_v, [c16])
            t0 = found == 0
            valid = jnp.logical_or(lanes < found, t0)
            jv = jnp.where(t0, lanes, jv)
            vals = jnp.where(valid, jnp.where(t0, 0.0, vals), -3.0e38)
            mx = jnp.max(vals)
            e = jnp.where(valid, jnp.exp(vals - mx), 0.0)
            s = jnp.sum(e)
            jsel_v[pl.ds(ri * 16, 16)] = jv + b * n
            wsel_v[pl.ds(ri * 16, 16)] = e / s

        nch = rows_per // RCH
        row0 = wid * rows_per

        def sel_chunk(g, carry):
            for sub in range(2):
                buf = pch[sub]

                @pl.when(2 * g + sub + 1 < nch)
                def _():
                    nxt = row0 + (2 * g + sub + 1) * RCH
                    pltpu.async_copy(padj_hbm.at[pl.ds(nxt, RCH)],
                                     pch[1 - sub], sem_p[1 - sub])
                pltpu.make_async_copy(padj_hbm.at[pl.ds(0, RCH)],
                                      buf, sem_p[sub]).wait()
                for r in range(RCH):
                    select_row(buf.at[r], (2 * g + sub) * RCH + r)
            return carry

        pltpu.async_copy(padj_hbm.at[pl.ds(row0, RCH)], pch0_v, sem_p0)
        lax.fori_loop(0, nch // 2, sel_chunk, jnp.int32(0))

        def issue(ri, buf):
            jrb[buf][...] = jsel_v[pl.ds(ri * 16, 16)]
            pltpu.async_copy(sup_hbm.at[jrb[buf]], nbb[buf], sem_s[buf])
            pltpu.async_copy(feat_hbm.at[b * n + row0 + ri],
                             frb[buf], sem_f[buf])

        def agg_row(ri, buf):
            pltpu.make_async_copy(sup_hbm.at[jrb[buf]], nbb[buf],
                                  sem_s[buf]).wait()
            pltpu.make_async_copy(feat_hbm.at[0], frb[buf],
                                  sem_f[buf]).wait()
            w16 = wsel_v[pl.ds(ri * 16, 16)]
            wk = [w16[jnp.full((16,), k, jnp.int32)] for k in range(K_NB)]
            nb_v = nbb[buf]
            frow_v = frb[buf]
            orow_v = orb[buf]

            @pl.when(ri >= 2)
            def _():
                pltpu.make_async_copy(orow_v, out_hbm.at[0],
                                      sem_o[buf]).wait()
            for dv in range(d // 16):
                dsl = pl.ds(dv * 16, 16)
                accs = [wk[c] * nb_v[c, dsl] for c in range(4)]
                for k in range(4, K_NB):
                    c = k % 4
                    accs[c] = accs[c] + wk[k] * nb_v[k, dsl]
                acc = (accs[0] + accs[1]) + (accs[2] + accs[3])
                o = ALPHA * frow_v[dsl] + (1.0 - ALPHA) * acc + bias_v[dsl]
                orow_v[dsl] = jnp.maximum(o, 0.0)

            @pl.when(ri + 2 < rows_per)
            def _():
                issue(ri + 2, buf)
            pltpu.async_copy(orow_v, out_hbm.at[b * n + row0 + ri],
                             sem_o[buf])

        issue(jnp.int32(0), 0)
        issue(jnp.int32(1), 1)

        def agg_pair(g, carry):
            agg_row(2 * g, 0)
            agg_row(2 * g + 1, 1)
            return carry
        lax.fori_loop(0, rows_per // 2, agg_pair, jnp.int32(0))

        pltpu.make_async_copy(or0_v, out_hbm.at[0], sem_o0).wait()
        pltpu.make_async_copy(or1_v, out_hbm.at[0], sem_o1).wait()
        return carry0

    lax.fori_loop(0, b_total, batch_fn, jnp.int32(0))


def _tc_stage(features, kernel, attn_kernel):
    b_total, n, f = features.shape
    d = kernel.shape[1]
    bn = 512

    sup, imp = pl.pallas_call(
        _matmul_body,
        grid=(b_total, n // bn),
        in_specs=[
            pl.BlockSpec((1, bn, f), lambda b, nb: (b, nb, 0)),
            pl.BlockSpec((f, d), lambda b, nb: (0, 0)),
            pl.BlockSpec((f, 1), lambda b, nb: (0, 0)),
        ],
        out_specs=[
            pl.BlockSpec((1, bn, d), lambda b, nb: (b, nb, 0)),
            pl.BlockSpec((1, bn, 1), lambda b, nb: (b, nb, 0)),
        ],
        out_shape=[
            jax.ShapeDtypeStruct((b_total, n, d), jnp.float32),
            jax.ShapeDtypeStruct((b_total, n, 1), jnp.float32),
        ],
    )(features, kernel, attn_kernel)

    nchunks = n // RANK_CHUNK
    imp3 = imp.reshape(b_total, 1, n)
    rank = pl.pallas_call(
        _rank_body,
        grid=(b_total, nchunks),
        in_specs=[pl.BlockSpec((1, 1, n), lambda b, jb: (b, 0, 0))],
        out_specs=pl.BlockSpec((1, 1, RANK_CHUNK),
                               lambda b, jb: (b * nchunks + jb, 0, 0)),
        out_shape=jax.ShapeDtypeStruct((b_total * nchunks, 1, RANK_CHUNK),
                                       jnp.int32),
    )(imp3).reshape(b_total, n)
    return sup, imp, rank


def _pack_adj(adj):
    n = adj.shape[0]
    nw = n // 32
    bn = 512
    col = jnp.arange(n)
    word = col // 32
    bit = col % 32
    tgt = jnp.where(bit < 16, word, nw + word)
    pmat = (jnp.zeros((n, 2 * nw), jnp.float32)
            .at[col, tgt].set(jnp.exp2((bit % 16).astype(jnp.float32))))
    pmat = pmat.astype(jnp.bfloat16)
    return pl.pallas_call(
        _pack_body,
        grid=(n // bn,),
        in_specs=[
            pl.BlockSpec((bn, n), lambda i: (i, 0)),
            pl.BlockSpec((n, 2 * nw), lambda i: (0, 0)),
        ],
        out_specs=pl.BlockSpec((bn, nw), lambda i: (i, 0)),
        out_shape=jax.ShapeDtypeStruct((n, nw), jnp.int32),
    )(adj, pmat)


def _sc_stage(adj, features, sup, imp, rank, bias):
    b_total, n, f = features.shape
    d = sup.shape[2]
    padj = _pack_adj(adj)
    try:
        info = plsc.get_sparse_core_info()
        nc, ns = info.num_cores, info.num_subcores
    except Exception:
        nc, ns = 2, 16

    nwords = n // 32
    body = functools.partial(_sc_body, nc, ns, n, b_total, d)
    sc = pl.kernel(
        body,
        out_type=jax.ShapeDtypeStruct((b_total * n, d), jnp.float32),
        mesh=plsc.VectorSubcoreMesh(core_axis_name="c", subcore_axis_name="s"),
        compiler_params=pltpu.CompilerParams(needs_layout_passes=False),
        scratch_types=[
            pltpu.VMEM((n,), jnp.int32),
            pltpu.VMEM((n,), jnp.float32),
            pltpu.VMEM((n,), jnp.int32),
            pltpu.VMEM((n,), jnp.float32),
            pltpu.VMEM((d,), jnp.float32),
            pltpu.VMEM((n // (nc * ns) * 16,), jnp.int32),
            pltpu.VMEM((n // (nc * ns) * 16,), jnp.float32),
            pltpu.VMEM((16,), jnp.int32),
            pltpu.VMEM((RCH, nwords), jnp.int32),
            pltpu.VMEM((RCH, nwords), jnp.int32),
            pltpu.VMEM((16,), jnp.int32),
            pltpu.VMEM((16,), jnp.int32),
            pltpu.VMEM((16, d), jnp.float32),
            pltpu.VMEM((16, d), jnp.float32),
            pltpu.VMEM((d,), jnp.float32),
            pltpu.VMEM((d,), jnp.float32),
            pltpu.VMEM((d,), jnp.float32),
            pltpu.VMEM((d,), jnp.float32),
            pltpu.SemaphoreType.DMA, pltpu.SemaphoreType.DMA,
            pltpu.SemaphoreType.DMA, pltpu.SemaphoreType.DMA,
            pltpu.SemaphoreType.DMA, pltpu.SemaphoreType.DMA,
            pltpu.SemaphoreType.DMA, pltpu.SemaphoreType.DMA,
        ],
    )
    out_flat = sc(padj,
                  features.reshape(b_total * n, f),
                  sup.reshape(b_total * n, d),
                  imp.reshape(b_total, n),
                  rank,
                  bias)
    return out_flat.reshape(b_total, n, d)


def kernel(adj, features, attn_kernel, kernel, bias):
    sup, imp, rank = _tc_stage(features, kernel, attn_kernel)
    return _sc_stage(adj, features, sup, imp, rank, bias)


# speedup vs baseline: 1.2375x; 1.0585x over previous
import functools

import jax
import jax.numpy as jnp
from jax import lax
from jax.experimental import pallas as pl
from jax.experimental.pallas import tpu as pltpu
from jax.experimental.pallas import tpu_sc as plsc

K_NB = 16
ALPHA = 0.5
PRE = 64
RANK_CHUNK = 256
RCH = 8


def _matmul_body(feat_ref, ker_ref, attn_ref, sup_ref, imp_ref):
    f = feat_ref[0]
    sup_ref[0] = jnp.dot(f, ker_ref[...], preferred_element_type=jnp.float32)
    imp_ref[0] = jnp.dot(f, attn_ref[...], preferred_element_type=jnp.float32)


def _rank_body(imp_ref, rank_ref):
    jb = pl.program_id(1)
    n = imp_ref.shape[2]
    base = jb * RANK_CHUNK
    row = imp_ref[0, 0, :]
    vj = imp_ref[0, 0, pl.ds(base, RANK_CHUNK)]
    vj2 = vj[:, None]
    jidx = base + lax.broadcasted_iota(jnp.int32, (RANK_CHUNK, 1), 0)
    jp = lax.broadcasted_iota(jnp.int32, (RANK_CHUNK, n), 1)
    rowb = row[None, :]
    gt = (rowb > vj2).astype(jnp.int32)
    eqb = ((rowb == vj2) & (jp < jidx)).astype(jnp.int32)
    rank_ref[0, 0, :] = jnp.sum(gt + eqb, axis=1)


def _pack_body(adj_ref, pmat_ref, out_ref):
    h = jnp.dot(adj_ref[...].astype(jnp.bfloat16), pmat_ref[...],
                preferred_element_type=jnp.float32)
    nw = out_ref.shape[1]
    lo = h[:, :nw].astype(jnp.int32)
    hi = h[:, nw:].astype(jnp.int32)
    out_ref[...] = lo | (hi << 16)


def _sc_body(nc, ns, n, b_total, d,
             padj_hbm, feat_hbm, sup_hbm, imp_hbm, rank_hbm, bias_hbm,
             out_hbm,
             perm_v, simp_v, rank_v, impb_v, bias_v, jsel_v, wsel_v,
             slots_v, pch0_v, pch1_v,
             jr0_v, jr1_v, nb0_v, nb1_v,
             fr0_v, fr1_v, or0_v, or1_v,
             sem_p0, sem_p1, sem_s0, sem_s1, sem_f0, sem_f1,
             sem_o0, sem_o1):
    nw = n // 32
    rows_per = n // (nc * ns)
    cid = lax.axis_index("c")
    sid = lax.axis_index("s")
    wid = sid * nc + cid
    lanes = lax.iota(jnp.int32, 16)
    pch = (pch0_v, pch1_v)
    sem_p = (sem_p0, sem_p1)
    jrb = (jr0_v, jr1_v)
    nbb = (nb0_v, nb1_v)
    frb = (fr0_v, fr1_v)
    orb = (or0_v, or1_v)
    sem_s = (sem_s0, sem_s1)
    sem_f = (sem_f0, sem_f1)
    sem_o = (sem_o0, sem_o1)

    pltpu.sync_copy(bias_hbm, bias_v)

    def batch_fn(b, carry0):
        row0 = wid * rows_per
        pltpu.sync_copy(imp_hbm.at[b], impb_v)
        pltpu.sync_copy(rank_hbm.at[b], rank_v)

        def build(i, carry):
            sl = pl.ds(i * 16, 16)
            r = rank_v[sl]
            jv = i * 16 + lanes
            plsc.store_scatter(perm_v, [r], jv)
            plsc.store_scatter(simp_v, [r], impb_v[sl])
            return carry
        lax.fori_loop(0, n // 16, build, jnp.int32(0))

        row0 = wid * rows_per

        def scan128(prow_ref, c_base, found):
            for v in range(PRE // 16):
                pv = perm_v[pl.ds(c_base + v * 16, 16)]
                word = plsc.load_gather(prow_ref, [pv >> 5])
                bit = (word >> (pv & 31)) & 1
                m = bit != 0
                mi = bit
                cpos = plsc.cumsum(mi) + found
                sel = jnp.logical_and(m, cpos <= K_NB)
                slot = jnp.clip(cpos - 1, 0, 15)
                cvec = c_base + v * 16 + lanes
                plsc.store_scatter(slots_v, [slot], cvec, mask=sel)
                found = cpos[15]
            return found

        def select_row(prow_ref, ri):
            slots_v[...] = jnp.zeros((16,), jnp.int32)

            def cond(st):
                c_base, fnd = st
                return jnp.logical_and(fnd < K_NB, c_base < n)

            def more(st):
                c_base, fnd = st
                fnd = scan128(prow_ref, c_base, fnd)
                return (c_base + PRE, fnd)

            _, found = lax.while_loop(cond, more,
                                      (jnp.int32(0), jnp.int32(0)))

            c16 = slots_v[...]
            jv = plsc.load_gather(perm_v, [c16])
            vals = plsc.load_gather(simp_v, [c16])
            t0 = found == 0
            valid = jnp.logical_or(lanes < found, t0)
            jv = jnp.where(t0, lanes, jv)
            vals = jnp.where(valid, jnp.where(t0, 0.0, vals), -3.0e38)
            mx = jnp.max(vals)
            e = jnp.where(valid, jnp.exp(vals - mx), 0.0)
            s = jnp.sum(e)
            jsel_v[pl.ds(ri * 16, 16)] = jv + b * n
            wsel_v[pl.ds(ri * 16, 16)] = e / s

        nch = rows_per // RCH
        row0 = wid * rows_per

        def sel_chunk(g, carry):
            for sub in range(2):
                buf = pch[sub]

                @pl.when(2 * g + sub + 1 < nch)
                def _():
                    nxt = row0 + (2 * g + sub + 1) * RCH
                    pltpu.async_copy(padj_hbm.at[pl.ds(nxt, RCH)],
                                     pch[1 - sub], sem_p[1 - sub])
                pltpu.make_async_copy(padj_hbm.at[pl.ds(0, RCH)],
                                      buf, sem_p[sub]).wait()
                for r in range(RCH):
                    select_row(buf.at[r], (2 * g + sub) * RCH + r)
            return carry

        pltpu.async_copy(padj_hbm.at[pl.ds(row0, RCH)], pch0_v, sem_p0)
        lax.fori_loop(0, nch // 2, sel_chunk, jnp.int32(0))

        def issue(ri, buf):
            jrb[buf][...] = jsel_v[pl.ds(ri * 16, 16)]
            pltpu.async_copy(sup_hbm.at[jrb[buf]], nbb[buf], sem_s[buf])
            pltpu.async_copy(feat_hbm.at[b * n + row0 + ri],
                             frb[buf], sem_f[buf])

        def agg_row(ri, buf):
            pltpu.make_async_copy(sup_hbm.at[jrb[buf]], nbb[buf],
                                  sem_s[buf]).wait()
            pltpu.make_async_copy(feat_hbm.at[0], frb[buf],
                                  sem_f[buf]).wait()
            w16 = wsel_v[pl.ds(ri * 16, 16)]
            wk = [w16[jnp.full((16,), k, jnp.int32)] for k in range(K_NB)]
            nb_v = nbb[buf]
            frow_v = frb[buf]
            orow_v = orb[buf]

            @pl.when(ri >= 2)
            def _():
                pltpu.make_async_copy(orow_v, out_hbm.at[0],
                                      sem_o[buf]).wait()
            for dv in range(d // 16):
                dsl = pl.ds(dv * 16, 16)
                accs = [wk[c] * nb_v[c, dsl] for c in range(4)]
                for k in range(4, K_NB):
                    c = k % 4
                    accs[c] = accs[c] + wk[k] * nb_v[k, dsl]
                acc = (accs[0] + accs[1]) + (accs[2] + accs[3])
                o = ALPHA * frow_v[dsl] + (1.0 - ALPHA) * acc + bias_v[dsl]
                orow_v[dsl] = jnp.maximum(o, 0.0)

            @pl.when(ri + 2 < rows_per)
            def _():
                issue(ri + 2, buf)
            pltpu.async_copy(orow_v, out_hbm.at[b * n + row0 + ri],
                             sem_o[buf])

        issue(jnp.int32(0), 0)
        issue(jnp.int32(1), 1)

        def agg_pair(g, carry):
            agg_row(2 * g, 0)
            agg_row(2 * g + 1, 1)
            return carry
        lax.fori_loop(0, rows_per // 2, agg_pair, jnp.int32(0))

        pltpu.make_async_copy(or0_v, out_hbm.at[0], sem_o0).wait()
        pltpu.make_async_copy(or1_v, out_hbm.at[0], sem_o1).wait()
        return carry0

    lax.fori_loop(0, b_total, batch_fn, jnp.int32(0))


def _tc_stage(features, kernel, attn_kernel):
    b_total, n, f = features.shape
    d = kernel.shape[1]
    bn = 512

    sup, imp = pl.pallas_call(
        _matmul_body,
        grid=(b_total, n // bn),
        in_specs=[
            pl.BlockSpec((1, bn, f), lambda b, nb: (b, nb, 0)),
            pl.BlockSpec((f, d), lambda b, nb: (0, 0)),
            pl.BlockSpec((f, 1), lambda b, nb: (0, 0)),
        ],
        out_specs=[
            pl.BlockSpec((1, bn, d), lambda b, nb: (b, nb, 0)),
            pl.BlockSpec((1, bn, 1), lambda b, nb: (b, nb, 0)),
        ],
        out_shape=[
            jax.ShapeDtypeStruct((b_total, n, d), jnp.float32),
            jax.ShapeDtypeStruct((b_total, n, 1), jnp.float32),
        ],
    )(features, kernel, attn_kernel)

    nchunks = n // RANK_CHUNK
    imp3 = imp.reshape(b_total, 1, n)
    rank = pl.pallas_call(
        _rank_body,
        grid=(b_total, nchunks),
        in_specs=[pl.BlockSpec((1, 1, n), lambda b, jb: (b, 0, 0))],
        out_specs=pl.BlockSpec((1, 1, RANK_CHUNK),
                               lambda b, jb: (b * nchunks + jb, 0, 0)),
        out_shape=jax.ShapeDtypeStruct((b_total * nchunks, 1, RANK_CHUNK),
                                       jnp.int32),
    )(imp3).reshape(b_total, n)
    return sup, imp, rank


def _pack_adj(adj):
    import numpy as np
    n = adj.shape[0]
    nw = n // 32
    bn = 512
    col = np.arange(n)
    word = col // 32
    bit = col % 32
    tgt = np.where(bit < 16, word, nw + word)
    pmat_np = np.zeros((n, 2 * nw), np.float32)
    pmat_np[col, tgt] = np.exp2(bit % 16)
    pmat = jnp.asarray(pmat_np.astype(jnp.bfloat16))
    return pl.pallas_call(
        _pack_body,
        grid=(n // bn,),
        in_specs=[
            pl.BlockSpec((bn, n), lambda i: (i, 0)),
            pl.BlockSpec((n, 2 * nw), lambda i: (0, 0)),
        ],
        out_specs=pl.BlockSpec((bn, nw), lambda i: (i, 0)),
        out_shape=jax.ShapeDtypeStruct((n, nw), jnp.int32),
    )(adj, pmat)


def _sc_stage(adj, features, sup, imp, rank, bias):
    b_total, n, f = features.shape
    d = sup.shape[2]
    padj = _pack_adj(adj)
    try:
        info = plsc.get_sparse_core_info()
        nc, ns = info.num_cores, info.num_subcores
    except Exception:
        nc, ns = 2, 16

    nwords = n // 32
    body = functools.partial(_sc_body, nc, ns, n, b_total, d)
    sc = pl.kernel(
        body,
        out_type=jax.ShapeDtypeStruct((b_total * n, d), jnp.float32),
        mesh=plsc.VectorSubcoreMesh(core_axis_name="c", subcore_axis_name="s"),
        compiler_params=pltpu.CompilerParams(needs_layout_passes=False),
        scratch_types=[
            pltpu.VMEM((n,), jnp.int32),
            pltpu.VMEM((n,), jnp.float32),
            pltpu.VMEM((n,), jnp.int32),
            pltpu.VMEM((n,), jnp.float32),
            pltpu.VMEM((d,), jnp.float32),
            pltpu.VMEM((n // (nc * ns) * 16,), jnp.int32),
            pltpu.VMEM((n // (nc * ns) * 16,), jnp.float32),
            pltpu.VMEM((16,), jnp.int32),
            pltpu.VMEM((RCH, nwords), jnp.int32),
            pltpu.VMEM((RCH, nwords), jnp.int32),
            pltpu.VMEM((16,), jnp.int32),
            pltpu.VMEM((16,), jnp.int32),
            pltpu.VMEM((16, d), jnp.float32),
            pltpu.VMEM((16, d), jnp.float32),
            pltpu.VMEM((d,), jnp.float32),
            pltpu.VMEM((d,), jnp.float32),
            pltpu.VMEM((d,), jnp.float32),
            pltpu.VMEM((d,), jnp.float32),
            pltpu.SemaphoreType.DMA, pltpu.SemaphoreType.DMA,
            pltpu.SemaphoreType.DMA, pltpu.SemaphoreType.DMA,
            pltpu.SemaphoreType.DMA, pltpu.SemaphoreType.DMA,
            pltpu.SemaphoreType.DMA, pltpu.SemaphoreType.DMA,
        ],
    )
    out_flat = sc(padj,
                  features.reshape(b_total * n, f),
                  sup.reshape(b_total * n, d),
                  imp.reshape(b_total, n),
                  rank,
                  bias)
    return out_flat.reshape(b_total, n, d)


def kernel(adj, features, attn_kernel, kernel, bias):
    sup, imp, rank = _tc_stage(features, kernel, attn_kernel)
    return _sc_stage(adj, features, sup, imp, rank, bias)
